# Initial kernel scaffold; baseline (speedup 1.0000x reference)
import jax
import jax.numpy as jnp
from jax.experimental import pallas as pl


def kernel(vehicle_ids, tokenized_prompts, token_table, cls_ctx):
    raise NotImplementedError("write your pallas kernel here")


# speedup vs baseline: 1.0556x; 1.0556x over previous
import functools

import jax
import jax.numpy as jnp
from jax import lax
from jax.experimental import pallas as pl
from jax.experimental.pallas import tpu as pltpu
from jax.experimental.pallas import tpu_sc as plsc

_NUM_CLASS = 13164
_VOCAB = 49408
_D = 512
_SEQ = 77
_B = 1024
_PRE = 12
_SUF = 20
_NTOK = _PRE + _SUF
_NCLS = 4
_NZERO = _SEQ - _NTOK - _NCLS

_NC, _NS = 2, 16
_NW = _NC * _NS
_BPW = _B // _NW


def _body(tok_idx_hbm, cls_idx_hbm, table_hbm, cls_hbm, zeros_hbm, out_hbm,
          idx_v, cidx_v, rows_v, cls_v, zero_v, gsem, csem):
    wid = lax.axis_index("s") * _NC + lax.axis_index("c")
    base = wid * _BPW

    pltpu.sync_copy(tok_idx_hbm.at[pl.ds(base, _BPW)], idx_v)
    pltpu.sync_copy(cls_idx_hbm.at[pl.ds(base * _NCLS, _BPW * _NCLS)], cidx_v)
    pltpu.sync_copy(zeros_hbm, zero_v)

    pltpu.async_copy(cls_hbm.at[cidx_v], cls_v, csem).wait()

    def step(i, _):
        b = base + i
        o = b * _SEQ
        pltpu.async_copy(table_hbm.at[idx_v.at[i]], rows_v, gsem).wait()
        pltpu.sync_copy(rows_v.at[pl.ds(0, _PRE)], out_hbm.at[pl.ds(o, _PRE)])
        pltpu.sync_copy(cls_v.at[pl.ds(i * _NCLS, _NCLS)],
                        out_hbm.at[pl.ds(o + _PRE, _NCLS)])
        pltpu.sync_copy(rows_v.at[pl.ds(_PRE, _SUF)],
                        out_hbm.at[pl.ds(o + _PRE + _NCLS, _SUF)])
        pltpu.sync_copy(zero_v, out_hbm.at[pl.ds(o + _NTOK + _NCLS, _NZERO)])
        return 0

    lax.fori_loop(0, _BPW, step, 0)


@jax.jit
def kernel(vehicle_ids, tokenized_prompts, token_table, cls_ctx):
    tok_idx = jnp.concatenate(
        [tokenized_prompts[:, :_PRE], tokenized_prompts[:, _SEQ - _SUF:]],
        axis=1).astype(jnp.int32)
    cls_idx = (vehicle_ids[:, None].astype(jnp.int32) * _NCLS
               + jnp.arange(_NCLS, dtype=jnp.int32)).reshape(-1)
    cls_flat = cls_ctx.reshape(_NUM_CLASS * _NCLS, _D)
    zeros = jnp.zeros((_NZERO, _D), jnp.float32)

    mesh = plsc.VectorSubcoreMesh(core_axis_name="c", subcore_axis_name="s",
                                  num_cores=_NC, num_subcores=_NS)
    run = pl.kernel(
        _body,
        out_type=jax.ShapeDtypeStruct((_B * _SEQ, _D), jnp.float32),
        mesh=mesh,
        scratch_types=[
            pltpu.VMEM((_BPW, _NTOK), jnp.int32),
            pltpu.VMEM((_BPW * _NCLS,), jnp.int32),
            pltpu.VMEM((_NTOK, _D), jnp.float32),
            pltpu.VMEM((_BPW * _NCLS, _D), jnp.float32),
            pltpu.VMEM((_NZERO, _D), jnp.float32),
            pltpu.SemaphoreType.DMA,
            pltpu.SemaphoreType.DMA,
        ],
        compiler_params=pltpu.CompilerParams(use_tc_tiling_on_sc=False),
    )
    out = run(tok_idx, cls_idx, token_table, cls_flat, zeros)
    return out.reshape(_B, _SEQ, _D)


# speedup vs baseline: 1.1009x; 1.0429x over previous
import functools

import jax
import jax.numpy as jnp
from jax import lax
from jax.experimental import pallas as pl
from jax.experimental.pallas import tpu as pltpu
from jax.experimental.pallas import tpu_sc as plsc

_NUM_CLASS = 13164
_VOCAB = 49408
_D = 512
_SEQ = 77
_B = 1024
_PRE = 12
_SUF = 20
_NTOK = _PRE + _SUF
_NCLS = 4
_NZERO = _SEQ - _NTOK - _NCLS

_NC, _NS = 2, 16
_NW = _NC * _NS
_BPW = _B // _NW


def _body(tok_idx_hbm, cls_idx_hbm, table_hbm, cls_hbm, zeros_hbm, out_hbm,
          idx_v, cidx_v, rows_v, cls_v, zero_v, gsem0, gsem1, csem):
    wid = lax.axis_index("s") * _NC + lax.axis_index("c")
    base = wid * _BPW

    pltpu.sync_copy(tok_idx_hbm.at[pl.ds(base, _BPW)], idx_v)
    pltpu.sync_copy(cls_idx_hbm.at[pl.ds(base * _NCLS, _BPW * _NCLS)], cidx_v)
    pltpu.sync_copy(zeros_hbm, zero_v)

    pltpu.async_copy(cls_hbm.at[cidx_v], cls_v, csem).wait()

    gsems = (gsem0, gsem1)

    def fire(i, k):
        pltpu.async_copy(table_hbm.at[idx_v.at[i]], rows_v.at[k], gsems[k])

    def drain_and_write(i, k):
        b = base + i
        o = b * _SEQ
        pltpu.make_async_copy(table_hbm.at[idx_v.at[i]],
                              rows_v.at[k], gsems[k]).wait()
        buf = rows_v.at[k]
        pltpu.sync_copy(buf.at[pl.ds(0, _PRE)], out_hbm.at[pl.ds(o, _PRE)])
        pltpu.sync_copy(cls_v.at[pl.ds(i * _NCLS, _NCLS)],
                        out_hbm.at[pl.ds(o + _PRE, _NCLS)])
        pltpu.sync_copy(buf.at[pl.ds(_PRE, _SUF)],
                        out_hbm.at[pl.ds(o + _PRE + _NCLS, _SUF)])
        pltpu.sync_copy(zero_v, out_hbm.at[pl.ds(o + _NTOK + _NCLS, _NZERO)])

    fire(0, 0)
    fire(1, 1)

    def step(j, _):
        drain_and_write(2 * j, 0)

        @pl.when(j < _BPW // 2 - 1)
        def _():
            fire(2 * j + 2, 0)

        drain_and_write(2 * j + 1, 1)

        @pl.when(j < _BPW // 2 - 1)
        def _():
            fire(2 * j + 3, 1)

        return 0

    lax.fori_loop(0, _BPW // 2, step, 0)


@jax.jit
def kernel(vehicle_ids, tokenized_prompts, token_table, cls_ctx):
    tok_idx = jnp.concatenate(
        [tokenized_prompts[:, :_PRE], tokenized_prompts[:, _SEQ - _SUF:]],
        axis=1).astype(jnp.int32)
    cls_idx = (vehicle_ids[:, None].astype(jnp.int32) * _NCLS
               + jnp.arange(_NCLS, dtype=jnp.int32)).reshape(-1)
    cls_flat = cls_ctx.reshape(_NUM_CLASS * _NCLS, _D)
    zeros = jnp.zeros((_NZERO, _D), jnp.float32)

    mesh = plsc.VectorSubcoreMesh(core_axis_name="c", subcore_axis_name="s",
                                  num_cores=_NC, num_subcores=_NS)
    run = pl.kernel(
        _body,
        out_type=jax.ShapeDtypeStruct((_B * _SEQ, _D), jnp.float32),
        mesh=mesh,
        scratch_types=[
            pltpu.VMEM((_BPW, _NTOK), jnp.int32),
            pltpu.VMEM((_BPW * _NCLS,), jnp.int32),
            pltpu.VMEM((2, _NTOK, _D), jnp.float32),
            pltpu.VMEM((_BPW * _NCLS, _D), jnp.float32),
            pltpu.VMEM((_NZERO, _D), jnp.float32),
            pltpu.SemaphoreType.DMA,
            pltpu.SemaphoreType.DMA,
            pltpu.SemaphoreType.DMA,
        ],
        compiler_params=pltpu.CompilerParams(use_tc_tiling_on_sc=False),
    )
    out = run(tok_idx, cls_idx, token_table, cls_flat, zeros)
    return out.reshape(_B, _SEQ, _D)


# speedup vs baseline: 2.4106x; 2.1896x over previous
import jax
import jax.numpy as jnp
from jax import lax
from jax.experimental import pallas as pl
from jax.experimental.pallas import tpu as pltpu
from jax.experimental.pallas import tpu_sc as plsc

_NUM_CLASS = 13164
_D = 512
_SEQ = 77
_B = 1024
_PRE = 12
_SUF = 20
_NCLS = 4
_ZSTART = 32
_NZERO = _SEQ - _ZSTART

_NC, _NS = 2, 16
_NW = _NC * _NS
_BPW = _B // _NW
_GRP = 4
_LANES = 16


def _idx_vec(idx_v, i, off):
    row = jnp.full((_LANES,), i, jnp.int32)
    col = lax.iota(jnp.int32, _LANES) + off
    return plsc.load_gather(idx_v, [row, col])


def _copy_rows(src, src_row, dst, dst_row, nrows):
    for r in range(nrows):
        for c in range(_D // _LANES):
            dst[dst_row + r, pl.ds(c * _LANES, _LANES)] = (
                src[src_row + r, pl.ds(c * _LANES, _LANES)])


def _body(idx_hbm, table_hbm, cls_hbm, zeros_hbm, out_hbm,
          idx_v, slab_v, suf_v, cls_v, gsem0, gsem1, csem0, csem1,
          wsem0, wsem1):
    wid = lax.axis_index("s") * _NC + lax.axis_index("c")
    base = wid * _BPW

    pltpu.sync_copy(idx_hbm.at[pl.ds(base, _BPW)], idx_v)
    pltpu.sync_copy(zeros_hbm, slab_v.at[0].at[pl.ds(_ZSTART, _NZERO)])
    pltpu.sync_copy(zeros_hbm, slab_v.at[1].at[pl.ds(_ZSTART, _NZERO)])

    gsems = (gsem0, gsem1)
    csems = (csem0, csem1)
    wsems = (wsem0, wsem1)

    def fire_gathers(i, k):
        slab = slab_v.at[k]
        pltpu.async_copy(table_hbm.at[_idx_vec(idx_v, i, 0)],
                         slab.at[pl.ds(0, 16)], gsems[k])
        pltpu.async_copy(table_hbm.at[_idx_vec(idx_v, i, 16)],
                         slab.at[pl.ds(16, 16)], gsems[k])
        pltpu.async_copy(table_hbm.at[_idx_vec(idx_v, i, 32)],
                         suf_v.at[k], gsems[k])
        vid = jnp.max(_idx_vec(idx_v, i, 48))
        pltpu.async_copy(cls_hbm.at[vid], cls_v.at[k], csems[k])

    def wait_gathers(i, k):
        slab = slab_v.at[k]
        pltpu.make_async_copy(table_hbm.at[_idx_vec(idx_v, i, 0)],
                              slab.at[pl.ds(0, 16)], gsems[k]).wait()
        pltpu.make_async_copy(table_hbm.at[_idx_vec(idx_v, i, 16)],
                              slab.at[pl.ds(16, 16)], gsems[k]).wait()
        pltpu.make_async_copy(table_hbm.at[_idx_vec(idx_v, i, 32)],
                              suf_v.at[k], gsems[k]).wait()
        vid = jnp.max(_idx_vec(idx_v, i, 48))
        pltpu.make_async_copy(cls_hbm.at[vid], cls_v.at[k],
                              csems[k]).wait()

    def fire_write(i, k):
        pltpu.async_copy(slab_v.at[k], out_hbm.at[base + i], wsems[k])

    def drain_write(i, k):
        pltpu.make_async_copy(slab_v.at[k], out_hbm.at[base + i],
                              wsems[k]).wait()

    fire_gathers(0, 0)
    fire_gathers(1, 1)

    def step(i, _):
        kb = lax.rem(i, 2)

        def batch(kb):
            wait_gathers(i, kb)
            _copy_rows(cls_v.at[kb], 0, slab_v.at[kb], _PRE, _NCLS)
            _copy_rows(suf_v.at[kb], 12, slab_v.at[kb], 32, 4)
            fire_write(i, kb)

            @pl.when(i >= 1)
            def _():
                drain_write(i - 1, 1 - kb)

                @pl.when(i <= _BPW - 2)
                def _():
                    fire_gathers(i + 1, 1 - kb)

        @pl.when(kb == 0)
        def _():
            batch(0)

        @pl.when(kb == 1)
        def _():
            batch(1)

        return 0

    lax.fori_loop(0, _BPW, step, 0)
    drain_write(_BPW - 1, 1)


@jax.jit
def kernel(vehicle_ids, tokenized_prompts, token_table, cls_ctx):
    p = tokenized_prompts.astype(jnp.int32)
    v = vehicle_ids.astype(jnp.int32)[:, None]
    idx = jnp.concatenate(
        [p[:, :_PRE], jnp.broadcast_to(p[:, 11:12], (_B, 4)),
         p[:, 57:73], p[:, 61:77],
         jnp.broadcast_to(v, (_B, 16))], axis=1)
    idx = jnp.pad(idx, ((0, 0), (0, 128 - 64)))
    zeros = jnp.zeros((_NZERO, _D), jnp.float32)

    mesh = plsc.VectorSubcoreMesh(core_axis_name="c", subcore_axis_name="s",
                                  num_cores=_NC, num_subcores=_NS)
    run = pl.kernel(
        _body,
        out_type=jax.ShapeDtypeStruct((_B, _SEQ, _D), jnp.float32),
        mesh=mesh,
        scratch_types=[
            pltpu.VMEM((_BPW, 128), jnp.int32),
            pltpu.VMEM((2, _SEQ, _D), jnp.float32),
            pltpu.VMEM((2, 16, _D), jnp.float32),
            pltpu.VMEM((2, _NCLS, _D), jnp.float32),
            pltpu.SemaphoreType.DMA,
            pltpu.SemaphoreType.DMA,
            pltpu.SemaphoreType.DMA,
            pltpu.SemaphoreType.DMA,
            pltpu.SemaphoreType.DMA,
            pltpu.SemaphoreType.DMA,
        ],
        compiler_params=pltpu.CompilerParams(needs_layout_passes=False),
    )
    return run(idx, token_table, cls_ctx, zeros)


# speedup vs baseline: 5.3237x; 2.2085x over previous
import jax
import jax.numpy as jnp
from jax import lax
from jax.experimental import pallas as pl
from jax.experimental.pallas import tpu as pltpu
from jax.experimental.pallas import tpu_sc as plsc

_NUM_CLASS = 13164
_D = 512
_SEQ = 77
_B = 1024
_PRE = 12
_SUF = 20
_NTOK = _PRE + _SUF
_NCLS = 4
_ZROWS = (_SEQ - 36) * _B

_NC, _NS = 2, 16
_NW = _NC * _NS
_TCH = 16
_TW = _B // _TCH
_CCH = 4
_CW = 8
_ZCH = 82
_ZW = 16
_L = 16


def _vec(ref, row, col):
    r = jnp.full((_L,), row, jnp.int32)
    c = jnp.full((_L,), col, jnp.int32)
    return plsc.load_gather(ref, [r, c])


def _idx16(ref, row, off):
    r = jnp.full((_L,), row, jnp.int32)
    c = lax.iota(jnp.int32, _L) + off
    return plsc.load_gather(ref, [r, c])


def _body(tokidx_hbm, vidc_hbm, table_hbm, cls_hbm, zeros_hbm, out_hbm,
          tokidx_v, vidc_v, tok_v, zbuf_v, stag_v, pos_v,
          gsem0, gsem1, wsem0, wsem1, csem, psem, zsem):
    wid = lax.axis_index("s") * _NC + lax.axis_index("c")

    pltpu.sync_copy(tokidx_hbm.at[wid], tokidx_v)
    pltpu.sync_copy(vidc_hbm.at[wid], vidc_v)
    pltpu.sync_copy(zeros_hbm, zbuf_v)

    s_out = wid + jnp.where(wid >= _PRE, _NCLS, 0)
    tbase = s_out * _B

    gsems = (gsem0, gsem1)
    wsems = (wsem0, wsem1)

    def fire_gathers(c, k):
        for m in range(_TW // _L):
            pltpu.async_copy(table_hbm.at[_idx16(tokidx_v, c, m * _L)],
                             tok_v.at[k].at[pl.ds(m * _L, _L)], gsems[k])

    def wait_gathers(c, k):
        for m in range(_TW // _L):
            pltpu.make_async_copy(table_hbm.at[_idx16(tokidx_v, c, m * _L)],
                                  tok_v.at[k].at[pl.ds(m * _L, _L)],
                                  gsems[k]).wait()

    def fire_write(c, k):
        pltpu.async_copy(tok_v.at[k], out_hbm.at[pl.ds(tbase + _TW * c, _TW)],
                         wsems[k])

    def drain_write(c, k):
        pltpu.make_async_copy(tok_v.at[k],
                              out_hbm.at[pl.ds(tbase + _TW * c, _TW)],
                              wsems[k]).wait()

    fire_gathers(0, 0)
    fire_gathers(1, 1)

    def tstep(c, _):
        def one(kb):
            wait_gathers(c, kb)
            fire_write(c, kb)

            @pl.when(c >= 1)
            def _():
                drain_write(c - 1, 1 - kb)

                @pl.when(c <= _TCH - 2)
                def _():
                    fire_gathers(c + 1, 1 - kb)

        @pl.when(lax.rem(c, 2) == 0)
        def _():
            one(0)

        @pl.when(lax.rem(c, 2) == 1)
        def _():
            one(1)

        return 0

    lax.fori_loop(0, _TCH, tstep, 0)
    drain_write(_TCH - 1, 1)

    def cstep(c, _):
        for g in range(_CW):
            v = jnp.max(_vec(vidc_v, c, g))
            pltpu.async_copy(cls_hbm.at[v], stag_v.at[g], csem)
        for g in range(_CW):
            v = jnp.max(_vec(vidc_v, c, g))
            pltpu.make_async_copy(cls_hbm.at[v], stag_v.at[g], csem).wait()

        @pl.when(c >= 1)
        def _():
            for j in range(_NCLS):
                pltpu.make_async_copy(
                    pos_v.at[j],
                    out_hbm.at[pl.ds(0, _CW)], psem).wait()

        for g in range(_CW):
            for j in range(_NCLS):
                for l in range(_D // _L):
                    pos_v[j, g, pl.ds(l * _L, _L)] = (
                        stag_v[g, j, pl.ds(l * _L, _L)])

        cb = _CW * (_CCH * wid + c)
        for j in range(_NCLS):
            pltpu.async_copy(pos_v.at[j],
                             out_hbm.at[pl.ds((_PRE + j) * _B + cb, _CW)],
                             psem)
        return 0

    lax.fori_loop(0, _CCH, cstep, 0)
    for j in range(_NCLS):
        pltpu.make_async_copy(pos_v.at[j], out_hbm.at[pl.ds(0, _CW)],
                              psem).wait()

    zbase = 36 * _B + (_ZCH * _ZW) * wid

    def zfire(z, _):
        pltpu.async_copy(zbuf_v, out_hbm.at[pl.ds(zbase + _ZW * z, _ZW)],
                         zsem)
        return 0

    lax.fori_loop(0, _ZCH, zfire, 0)

    def zdrain(z, _):
        pltpu.make_async_copy(zbuf_v, out_hbm.at[pl.ds(zbase, _ZW)],
                              zsem).wait()
        return 0

    lax.fori_loop(0, _ZCH, zdrain, 0)


@jax.jit
def kernel(vehicle_ids, tokenized_prompts, token_table, cls_ctx):
    p = tokenized_prompts.astype(jnp.int32)
    cols = list(range(_PRE)) + list(range(_SEQ - _SUF, _SEQ))
    tokidx = p[:, cols].T.reshape(_NW, _TCH, _TW)
    vidc = vehicle_ids.astype(jnp.int32).reshape(_NW, _CCH, _CW)
    vidc = jnp.pad(vidc, ((0, 0), (0, 0), (0, 128 - _CW)))
    zeros = jnp.zeros((_ZW, _D), jnp.float32)

    mesh = plsc.VectorSubcoreMesh(core_axis_name="c", subcore_axis_name="s",
                                  num_cores=_NC, num_subcores=_NS)
    run = pl.kernel(
        _body,
        out_type=jax.ShapeDtypeStruct((_SEQ * _B, _D), jnp.float32),
        mesh=mesh,
        scratch_types=[
            pltpu.VMEM((_TCH, _TW), jnp.int32),
            pltpu.VMEM((_CCH, 128), jnp.int32),
            pltpu.VMEM((2, _TW, _D), jnp.float32),
            pltpu.VMEM((_ZW, _D), jnp.float32),
            pltpu.VMEM((_CW, _NCLS, _D), jnp.float32),
            pltpu.VMEM((_NCLS, _CW, _D), jnp.float32),
            pltpu.SemaphoreType.DMA,
            pltpu.SemaphoreType.DMA,
            pltpu.SemaphoreType.DMA,
            pltpu.SemaphoreType.DMA,
            pltpu.SemaphoreType.DMA,
            pltpu.SemaphoreType.DMA,
            pltpu.SemaphoreType.DMA,
        ],
        compiler_params=pltpu.CompilerParams(needs_layout_passes=False),
    )
    out = run(tokidx, vidc, token_table, cls_ctx, zeros)
    return out.reshape(_SEQ, _B, _D).transpose(1, 0, 2)
